# baseline (speedup 1.0000x reference)
import functools

import jax
import jax.numpy as jnp
from jax import lax
from jax.experimental import pallas as pl
from jax.experimental.pallas import tpu as pltpu
from jax.experimental.pallas import tpu_sc as plsc

_mesh = plsc.VectorSubcoreMesh(core_axis_name="c", subcore_axis_name="s")


def _shuffle(v, idx):
    return v.at[idx].get(mode="promise_in_bounds")


@functools.partial(
    pl.kernel,
    mesh=_mesh,
    out_type=jax.ShapeDtypeStruct((16,), jnp.float32),
    scratch_types=[
        pltpu.VMEM((16,), jnp.float32),
        pltpu.VMEM((16,), jnp.float32),
    ],
)
def _sc_maxdiff(src_hbm, out_hbm, src_v, out_v):
    wid = lax.axis_index("s") * 2 + lax.axis_index("c")

    pltpu.sync_copy(src_hbm, src_v)
    x = src_v[...]
    pos = lax.broadcasted_iota(jnp.int32, (16,), 0)
    for s in (4, 2, 1):
        x = x + _shuffle(x, pos ^ s)
    y = _shuffle(x, jnp.where(pos == 7, 8, pos))
    base = jnp.where(pos < 15, 1.0, 0.0)
    add = jnp.where((pos == 1) | (pos == 7), y, 0.0)
    cpu = base + add
    gpu = base + add
    m = jnp.abs(cpu - gpu)
    for s in (8, 4, 2, 1):
        m = jnp.maximum(m, _shuffle(m, pos ^ s))
    out_v[...] = m

    @pl.when(wid == 0)
    def _():
        pltpu.sync_copy(out_v, out_hbm)


def kernel(src):
    flat = (
        jnp.zeros((16,), jnp.float32)
        .at[0:5]
        .set(src[0, :])
        .at[8:13]
        .set(src[1, :])
    )
    out = _sc_maxdiff(flat)
    return out[0] > 1e-06


# speedup vs baseline: 1.0921x; 1.0921x over previous
import functools

import jax
import jax.numpy as jnp
from jax import lax
from jax.experimental import pallas as pl
from jax.experimental.pallas import tpu as pltpu
from jax.experimental.pallas import tpu_sc as plsc

_mesh = plsc.VectorSubcoreMesh(
    core_axis_name="c", subcore_axis_name="s", num_cores=1
)


def _shuffle(v, idx):
    return v.at[idx].get(mode="promise_in_bounds")


@functools.partial(
    pl.kernel,
    mesh=_mesh,
    out_type=jax.ShapeDtypeStruct((16,), jnp.float32),
    scratch_types=[
        pltpu.VMEM((16,), jnp.float32),
        pltpu.VMEM((16,), jnp.float32),
    ],
)
def _sc_maxdiff(src_hbm, out_hbm, src_v, out_v):
    wid = lax.axis_index("s") * 2 + lax.axis_index("c")

    pltpu.sync_copy(src_hbm, src_v)
    x = src_v[...]
    pos = lax.broadcasted_iota(jnp.int32, (16,), 0)
    for s in (4, 2, 1):
        x = x + _shuffle(x, pos ^ s)
    y = _shuffle(x, jnp.where(pos == 7, 8, pos))
    base = jnp.where(pos < 15, 1.0, 0.0)
    add = jnp.where((pos == 1) | (pos == 7), y, 0.0)
    cpu = base + add
    gpu = base + add
    m = jnp.abs(cpu - gpu)
    for s in (8, 4, 2, 1):
        m = jnp.maximum(m, _shuffle(m, pos ^ s))
    out_v[...] = m

    @pl.when(wid == 0)
    def _():
        pltpu.sync_copy(out_v, out_hbm)


def kernel(src):
    flat = (
        jnp.zeros((16,), jnp.float32)
        .at[0:5]
        .set(src[0, :])
        .at[8:13]
        .set(src[1, :])
    )
    out = _sc_maxdiff(flat)
    return out[0] > 1e-06


# speedup vs baseline: 1.1247x; 1.0298x over previous
import functools

import jax
import jax.numpy as jnp
from jax import lax
from jax.experimental import pallas as pl
from jax.experimental.pallas import tpu as pltpu
from jax.experimental.pallas import tpu_sc as plsc

_mesh = plsc.VectorSubcoreMesh(
    core_axis_name="c", subcore_axis_name="s", num_cores=1, num_subcores=1
)


def _shuffle(v, idx):
    return v.at[idx].get(mode="promise_in_bounds")


@functools.partial(
    pl.kernel,
    mesh=_mesh,
    out_type=jax.ShapeDtypeStruct((16,), jnp.float32),
    scratch_types=[
        pltpu.VMEM((16,), jnp.float32),
        pltpu.VMEM((16,), jnp.float32),
    ],
)
def _sc_maxdiff(src_hbm, out_hbm, src_v, out_v):
    wid = lax.axis_index("s") * 2 + lax.axis_index("c")

    pltpu.sync_copy(src_hbm, src_v)
    x = src_v[...]
    pos = lax.broadcasted_iota(jnp.int32, (16,), 0)
    for s in (4, 2, 1):
        x = x + _shuffle(x, pos ^ s)
    y = _shuffle(x, jnp.where(pos == 7, 8, pos))
    base = jnp.where(pos < 15, 1.0, 0.0)
    add = jnp.where((pos == 1) | (pos == 7), y, 0.0)
    cpu = base + add
    gpu = base + add
    m = jnp.abs(cpu - gpu)
    for s in (8, 4, 2, 1):
        m = jnp.maximum(m, _shuffle(m, pos ^ s))
    out_v[...] = m

    @pl.when(wid == 0)
    def _():
        pltpu.sync_copy(out_v, out_hbm)


def kernel(src):
    flat = (
        jnp.zeros((16,), jnp.float32)
        .at[0:5]
        .set(src[0, :])
        .at[8:13]
        .set(src[1, :])
    )
    out = _sc_maxdiff(flat)
    return out[0] > 1e-06
